# baseline (speedup 1.0000x reference)
import jax
import jax.numpy as jnp
from jax import lax
from jax.experimental import pallas as pl
from jax.experimental.pallas import tpu as pltpu


def _round_up(x, m):
    return ((x + m - 1) // m) * m


def _pack_gates_ifog(w, h, hp, scale_ifo):
    i_, f_, g_, o_ = jnp.split(w, 4, axis=0)

    def pad(p):
        widths = [(0, hp - h)] + [(0, 0)] * (p.ndim - 1)
        return jnp.pad(p, widths)

    return jnp.concatenate(
        [scale_ifo * pad(i_), scale_ifo * pad(f_), scale_ifo * pad(o_), pad(g_)],
        axis=0)


def _make_body(hp, e_dim, bb, tc, t_total):
    t_exact = (t_total % tc == 0)

    def body(x_ref, wcat_ref, b_ref, wlin_ref, blin_ref,
             logits_ref, hlast_ref, lhs_sc, c_sc):
        chunk = pl.program_id(1)

        @pl.when(chunk == 0)
        def _init():
            lhs_sc[:, e_dim:] = jnp.zeros((bb, hp), jnp.bfloat16)
            c_sc[...] = jnp.zeros((bb, hp), jnp.float32)

        if not t_exact:
            steps = t_total - chunk * tc

        def step(i, carry):
            lhs_sc[:, :e_dim] = x_ref[i]
            gates = jnp.dot(lhs_sc[...], wcat_ref[...],
                            preferred_element_type=jnp.float32)
            gb = (gates + b_ref[...]).astype(jnp.bfloat16)
            t_all = jnp.tanh(gb)
            i_g = 0.5 * (t_all[:, 0 * hp:1 * hp] + 1.0)
            f_g = 0.5 * (t_all[:, 1 * hp:2 * hp] + 1.0)
            o_g = 0.5 * (t_all[:, 2 * hp:3 * hp] + 1.0)
            g_g = t_all[:, 3 * hp:4 * hp]
            c = c_sc[...]
            c_new = (f_g.astype(jnp.float32) * c
                     + i_g.astype(jnp.float32) * g_g.astype(jnp.float32))
            h_new = o_g * jnp.tanh(c_new.astype(jnp.bfloat16))
            if not t_exact:
                valid = i < steps
                h_new = jnp.where(valid, h_new, lhs_sc[:, e_dim:])
                c_new = jnp.where(valid, c_new, c)
            c_sc[...] = c_new
            lhs_sc[:, e_dim:] = h_new.astype(jnp.bfloat16)
            return carry

        lax.fori_loop(0, tc, step, 0, unroll=1)

        @pl.when(chunk == pl.num_programs(1) - 1)
        def _finalize():
            h_fin = lhs_sc[:, e_dim:]
            hlast_ref[...] = h_fin.astype(jnp.float32)
            logits_ref[...] = (
                jnp.dot(h_fin, wlin_ref[...], preferred_element_type=jnp.float32)
                + blin_ref[...])

    return body


def kernel(token_ids, embedding, w_ih, w_hh, b_ih, b_hh, w_lin, b_lin):
    B, T = token_ids.shape
    V, E = embedding.shape
    H = w_hh.shape[1]
    O = w_lin.shape[0]

    Hp = _round_up(H, 128)
    Op = _round_up(O, 128)
    B_pad = _round_up(B, 8)
    if B_pad <= 256:
        Bb = max(8, _round_up(B_pad // 2, 8))
    else:
        Bb = 128
    B_pad = _round_up(B_pad, Bb)

    tc = min(T, 32)
    n_chunks = -(-T // tc)
    T_pad = n_chunks * tc

    wx_T = _pack_gates_ifog(w_ih, H, Hp, 0.5).T
    wh_T = jnp.pad(_pack_gates_ifog(w_hh, H, Hp, 0.5),
                   ((0, 0), (0, Hp - H))).T
    wcat = jnp.concatenate([wx_T, wh_T], axis=0).astype(jnp.bfloat16)
    bias = (b_ih + b_hh)
    b_p = _pack_gates_ifog(bias, H, Hp, 0.5).reshape(1, 4 * Hp).astype(jnp.float32)

    wlin_p = jnp.pad(w_lin, ((0, Op - O), (0, Hp - H))).T.astype(jnp.bfloat16)
    blin_p = jnp.pad(b_lin, (0, Op - O)).reshape(1, Op).astype(jnp.float32)

    x = jnp.take(embedding.astype(jnp.bfloat16), token_ids.T, axis=0)
    if (T_pad - T) or (B_pad - B):
        x = jnp.pad(x, ((0, T_pad - T), (0, B_pad - B), (0, 0)))

    logits_pad, h_last_pad = pl.pallas_call(
        _make_body(Hp, E, Bb, tc, T),
        out_shape=(jax.ShapeDtypeStruct((B_pad, Op), jnp.float32),
                   jax.ShapeDtypeStruct((B_pad, Hp), jnp.float32)),
        grid_spec=pltpu.PrefetchScalarGridSpec(
            num_scalar_prefetch=0,
            grid=(B_pad // Bb, n_chunks),
            in_specs=[
                pl.BlockSpec((tc, Bb, E), lambda b, c: (c, b, 0)),
                pl.BlockSpec((E + Hp, 4 * Hp), lambda b, c: (0, 0),
                             pipeline_mode=pl.Buffered(1)),
                pl.BlockSpec((1, 4 * Hp), lambda b, c: (0, 0),
                             pipeline_mode=pl.Buffered(1)),
                pl.BlockSpec((Hp, Op), lambda b, c: (0, 0),
                             pipeline_mode=pl.Buffered(1)),
                pl.BlockSpec((1, Op), lambda b, c: (0, 0),
                             pipeline_mode=pl.Buffered(1)),
            ],
            out_specs=[pl.BlockSpec((Bb, Op), lambda b, c: (b, 0)),
                       pl.BlockSpec((Bb, Hp), lambda b, c: (b, 0))],
            scratch_shapes=[
                pltpu.VMEM((Bb, E + Hp), jnp.bfloat16),
                pltpu.VMEM((Bb, Hp), jnp.float32),
            ],
        ),
        compiler_params=pltpu.CompilerParams(
            dimension_semantics=("parallel", "arbitrary"),
            vmem_limit_bytes=48 * 1024 * 1024,
        ),
    )(x, wcat, b_p, wlin_p, blin_p)

    return logits_pad[:B, :O], h_last_pad[:B, :H]


# speedup vs baseline: 1.1252x; 1.1252x over previous
import jax
import jax.numpy as jnp
from jax import lax
from jax.experimental import pallas as pl
from jax.experimental.pallas import tpu as pltpu


def _round_up(x, m):
    return ((x + m - 1) // m) * m


def _pack_gates_ifog(w, h, hp, scale_ifo):
    i_, f_, g_, o_ = jnp.split(w, 4, axis=0)

    def pad(q):
        widths = [(0, hp - h)] + [(0, 0)] * (q.ndim - 1)
        return jnp.pad(q, widths)

    return jnp.concatenate(
        [scale_ifo * pad(i_), scale_ifo * pad(f_), scale_ifo * pad(o_), pad(g_)],
        axis=0)


def _make_body(hp, e_dim, bb, t_total):
    p = e_dim // 256
    S = bb + 1

    def gather_rows(ids_ref, table_ref, ystage, nb, t_row):
        for mi in range(bb):
            idx = pl.multiple_of(ids_ref[nb, t_row, mi], p)
            ystage[mi:mi + S * p:S, :] = table_ref[pl.ds(idx, p), :]

    def body(ids_ref, table_ref, wcat_ref, b_ref, wlin_ref, blin_ref,
             logits_ref, hlast_ref, ystage, h_sc, c_sc):
        nb = pl.program_id(0)
        h_sc[...] = jnp.zeros((bb, hp), jnp.bfloat16)
        c_sc[...] = jnp.zeros((bb, hp), jnp.float32)
        gather_rows(ids_ref, table_ref, ystage, nb, 0)

        def step(i, carry):
            pieces = []
            for j in range(p):
                yj = ystage[pl.ds(j * S, bb), :]
                pieces.append(
                    lax.bitcast_convert_type(yj << 16, jnp.float32)
                    .astype(jnp.bfloat16))
                pieces.append(
                    lax.bitcast_convert_type(yj & jnp.int32(-65536), jnp.float32)
                    .astype(jnp.bfloat16))
            pieces.append(h_sc[...])
            lhs = jnp.concatenate(pieces, axis=1)
            gates = jnp.dot(lhs, wcat_ref[...],
                            preferred_element_type=jnp.float32)
            gb = (gates + b_ref[...]).astype(jnp.bfloat16)
            t_all = jnp.tanh(gb)
            i_g = 0.5 * (t_all[:, 0 * hp:1 * hp] + 1.0)
            f_g = 0.5 * (t_all[:, 1 * hp:2 * hp] + 1.0)
            o_g = 0.5 * (t_all[:, 2 * hp:3 * hp] + 1.0)
            g_g = t_all[:, 3 * hp:4 * hp]
            c = c_sc[...]
            c_new = (f_g.astype(jnp.float32) * c
                     + i_g.astype(jnp.float32) * g_g.astype(jnp.float32))
            h_new = o_g * jnp.tanh(c_new.astype(jnp.bfloat16))
            c_sc[...] = c_new
            h_sc[...] = h_new.astype(jnp.bfloat16)
            gather_rows(ids_ref, table_ref, ystage, nb, i + 1)
            return carry

        lax.fori_loop(0, t_total, step, 0, unroll=1)

        h_fin = h_sc[...]
        hlast_ref[...] = h_fin.astype(jnp.float32)
        logits_ref[...] = (
            jnp.dot(h_fin, wlin_ref[...], preferred_element_type=jnp.float32)
            + blin_ref[...])

    return body


def kernel(token_ids, embedding, w_ih, w_hh, b_ih, b_hh, w_lin, b_lin):
    B, T = token_ids.shape
    V, E = embedding.shape
    H = w_hh.shape[1]
    O = w_lin.shape[0]

    Hp = _round_up(H, 128)
    Op = _round_up(O, 128)
    B_pad = _round_up(B, 8)
    if B_pad <= 256:
        Bb = max(8, _round_up(B_pad // 2, 8))
    else:
        Bb = 128
    B_pad = _round_up(B_pad, Bb)
    n_blocks = B_pad // Bb
    p = E // 256

    wx_T = _pack_gates_ifog(w_ih, H, Hp, 0.5).T
    perm = jnp.concatenate(
        [jnp.concatenate([jnp.arange(256 * j, 256 * (j + 1), 2),
                          jnp.arange(256 * j + 1, 256 * (j + 1), 2)])
         for j in range(p)])
    wx_T = wx_T[perm]
    wh_T = jnp.pad(_pack_gates_ifog(w_hh, H, Hp, 0.5),
                   ((0, 0), (0, Hp - H))).T
    wcat = jnp.concatenate([wx_T, wh_T], axis=0).astype(jnp.bfloat16)
    bias = (b_ih + b_hh)
    b_p = _pack_gates_ifog(bias, H, Hp, 0.5).reshape(1, 4 * Hp).astype(jnp.float32)

    wlin_p = jnp.pad(w_lin, ((0, Op - O), (0, Hp - H))).T.astype(jnp.bfloat16)
    blin_p = jnp.pad(b_lin, (0, Op - O)).reshape(1, Op).astype(jnp.float32)

    table = lax.bitcast_convert_type(
        embedding.astype(jnp.bfloat16).reshape(V, E // 2, 2), jnp.int32)
    table = table.reshape(V * p, 128)

    ids2 = jnp.pad(token_ids.T.astype(jnp.int32) * p,
                   ((0, 1), (0, B_pad - B)))
    ids3 = ids2.reshape(T + 1, n_blocks, Bb).transpose(1, 0, 2)

    logits_pad, h_last_pad = pl.pallas_call(
        _make_body(Hp, E, Bb, T),
        out_shape=(jax.ShapeDtypeStruct((B_pad, Op), jnp.float32),
                   jax.ShapeDtypeStruct((B_pad, Hp), jnp.float32)),
        grid_spec=pltpu.PrefetchScalarGridSpec(
            num_scalar_prefetch=1,
            grid=(n_blocks,),
            in_specs=[
                pl.BlockSpec((V * p, 128), lambda b, ids: (0, 0),
                             pipeline_mode=pl.Buffered(1)),
                pl.BlockSpec((E + Hp, 4 * Hp), lambda b, ids: (0, 0),
                             pipeline_mode=pl.Buffered(1)),
                pl.BlockSpec((1, 4 * Hp), lambda b, ids: (0, 0),
                             pipeline_mode=pl.Buffered(1)),
                pl.BlockSpec((Hp, Op), lambda b, ids: (0, 0),
                             pipeline_mode=pl.Buffered(1)),
                pl.BlockSpec((1, Op), lambda b, ids: (0, 0),
                             pipeline_mode=pl.Buffered(1)),
            ],
            out_specs=[pl.BlockSpec((Bb, Op), lambda b, ids: (b, 0)),
                       pl.BlockSpec((Bb, Hp), lambda b, ids: (b, 0))],
            scratch_shapes=[
                pltpu.VMEM(((Bb + 1) * p, 128), jnp.int32),
                pltpu.VMEM((Bb, Hp), jnp.bfloat16),
                pltpu.VMEM((Bb, Hp), jnp.float32),
            ],
        ),
        compiler_params=pltpu.CompilerParams(
            dimension_semantics=("parallel",),
            vmem_limit_bytes=48 * 1024 * 1024,
        ),
    )(ids3, table, wcat, b_p, wlin_p, blin_p)

    return logits_pad[:B, :O], h_last_pad[:B, :H]


# speedup vs baseline: 3.2493x; 2.8878x over previous
import jax
import jax.numpy as jnp
from jax import lax
from jax.experimental import pallas as pl
from jax.experimental.pallas import tpu as pltpu


def _round_up(x, m):
    return ((x + m - 1) // m) * m


def _pack_gates_ifog(w, h, hp, scale_ifo):
    i_, f_, g_, o_ = jnp.split(w, 4, axis=0)

    def pad(q):
        widths = [(0, hp - h)] + [(0, 0)] * (q.ndim - 1)
        return jnp.pad(q, widths)

    return jnp.concatenate(
        [scale_ifo * pad(i_), scale_ifo * pad(f_), scale_ifo * pad(o_), pad(g_)],
        axis=0)


def _make_pack_body(vb, e_dim, p):
    def body(in_ref, out_ref):
        x = in_ref[...]
        xi = lax.bitcast_convert_type(
            x.astype(jnp.bfloat16).astype(jnp.float32), jnp.int32)
        lo = xi[:, :e_dim // 2]
        hi = xi[:, e_dim // 2:]
        packed = ((lo >> 16) & jnp.int32(0xFFFF)) | (hi & jnp.int32(-65536))
        for j in range(p):
            out_ref[j:p * vb:p, :] = packed[:, 128 * j:128 * (j + 1)]
    return body


def _make_body(hp, e_dim, bb, t_total):
    p = e_dim // 256
    S = bb + 1

    def gather_rows(ids_ref, table_ref, ystage, nb, t_row):
        for mi in range(bb):
            idx = pl.multiple_of(ids_ref[nb, t_row, mi], p)
            ystage[mi:mi + S * p:S, :] = table_ref[pl.ds(idx, p), :]

    def body(ids_ref, table_ref, wcat_ref, b_ref, wlin_ref, blin_ref,
             logits_ref, hlast_ref, ystage, h_sc, c_sc):
        nb = pl.program_id(0)
        h_sc[...] = jnp.zeros((bb, hp), jnp.bfloat16)
        c_sc[...] = jnp.zeros((bb, hp), jnp.float32)
        gather_rows(ids_ref, table_ref, ystage, nb, 0)

        def step(i, carry):
            los, his = [], []
            for j in range(p):
                yj = ystage[pl.ds(j * S, bb), :]
                los.append(
                    lax.bitcast_convert_type(yj << 16, jnp.float32)
                    .astype(jnp.bfloat16))
                his.append(
                    lax.bitcast_convert_type(yj & jnp.int32(-65536), jnp.float32)
                    .astype(jnp.bfloat16))
            pieces = los + his + [h_sc[...]]
            lhs = jnp.concatenate(pieces, axis=1)
            gates = jnp.dot(lhs, wcat_ref[...],
                            preferred_element_type=jnp.float32)
            gb = (gates + b_ref[...]).astype(jnp.bfloat16)
            t_all = jnp.tanh(gb)
            i_g = 0.5 * (t_all[:, 0 * hp:1 * hp] + 1.0)
            f_g = 0.5 * (t_all[:, 1 * hp:2 * hp] + 1.0)
            o_g = 0.5 * (t_all[:, 2 * hp:3 * hp] + 1.0)
            g_g = t_all[:, 3 * hp:4 * hp]
            c = c_sc[...]
            c_new = (f_g.astype(jnp.float32) * c
                     + i_g.astype(jnp.float32) * g_g.astype(jnp.float32))
            h_new = o_g * jnp.tanh(c_new.astype(jnp.bfloat16))
            c_sc[...] = c_new
            h_sc[...] = h_new.astype(jnp.bfloat16)
            gather_rows(ids_ref, table_ref, ystage, nb, i + 1)
            return carry

        lax.fori_loop(0, t_total, step, 0, unroll=1)

        h_fin = h_sc[...]
        hlast_ref[...] = h_fin.astype(jnp.float32)
        logits_ref[...] = (
            jnp.dot(h_fin, wlin_ref[...], preferred_element_type=jnp.float32)
            + blin_ref[...])

    return body


def kernel(token_ids, embedding, w_ih, w_hh, b_ih, b_hh, w_lin, b_lin):
    B, T = token_ids.shape
    V, E = embedding.shape
    H = w_hh.shape[1]
    O = w_lin.shape[0]

    Hp = _round_up(H, 128)
    Op = _round_up(O, 128)
    B_pad = _round_up(B, 8)
    if B_pad <= 256:
        Bb = max(8, _round_up(B_pad // 2, 8))
    else:
        Bb = 128
    B_pad = _round_up(B_pad, Bb)
    n_blocks = B_pad // Bb
    p = E // 256

    wx_T = _pack_gates_ifog(w_ih, H, Hp, 0.5).T
    wh_T = jnp.pad(_pack_gates_ifog(w_hh, H, Hp, 0.5),
                   ((0, 0), (0, Hp - H))).T
    wcat = jnp.concatenate([wx_T, wh_T], axis=0).astype(jnp.bfloat16)
    bias = (b_ih + b_hh)
    b_p = _pack_gates_ifog(bias, H, Hp, 0.5).reshape(1, 4 * Hp).astype(jnp.float32)

    wlin_p = jnp.pad(w_lin, ((0, Op - O), (0, Hp - H))).T.astype(jnp.bfloat16)
    blin_p = jnp.pad(b_lin, (0, Op - O)).reshape(1, Op).astype(jnp.float32)

    vb = V
    for cand in range(min(V, 1024), 7, -8):
        if V % cand == 0:
            vb = cand
            break
    table = pl.pallas_call(
        _make_pack_body(vb, E, p),
        out_shape=jax.ShapeDtypeStruct((V * p, 128), jnp.int32),
        grid_spec=pltpu.PrefetchScalarGridSpec(
            num_scalar_prefetch=0,
            grid=(V // vb,),
            in_specs=[pl.BlockSpec((vb, E), lambda i: (i, 0))],
            out_specs=pl.BlockSpec((vb * p, 128), lambda i: (i, 0)),
        ),
        compiler_params=pltpu.CompilerParams(
            dimension_semantics=("parallel",),
        ),
    )(embedding)

    ids2 = jnp.pad(token_ids.T.astype(jnp.int32) * p,
                   ((0, 1), (0, B_pad - B)))
    ids3 = ids2.reshape(T + 1, n_blocks, Bb).transpose(1, 0, 2)

    logits_pad, h_last_pad = pl.pallas_call(
        _make_body(Hp, E, Bb, T),
        out_shape=(jax.ShapeDtypeStruct((B_pad, Op), jnp.float32),
                   jax.ShapeDtypeStruct((B_pad, Hp), jnp.float32)),
        grid_spec=pltpu.PrefetchScalarGridSpec(
            num_scalar_prefetch=1,
            grid=(n_blocks,),
            in_specs=[
                pl.BlockSpec((V * p, 128), lambda b, ids: (0, 0),
                             pipeline_mode=pl.Buffered(1)),
                pl.BlockSpec((E + Hp, 4 * Hp), lambda b, ids: (0, 0),
                             pipeline_mode=pl.Buffered(1)),
                pl.BlockSpec((1, 4 * Hp), lambda b, ids: (0, 0),
                             pipeline_mode=pl.Buffered(1)),
                pl.BlockSpec((Hp, Op), lambda b, ids: (0, 0),
                             pipeline_mode=pl.Buffered(1)),
                pl.BlockSpec((1, Op), lambda b, ids: (0, 0),
                             pipeline_mode=pl.Buffered(1)),
            ],
            out_specs=[pl.BlockSpec((Bb, Op), lambda b, ids: (b, 0)),
                       pl.BlockSpec((Bb, Hp), lambda b, ids: (b, 0))],
            scratch_shapes=[
                pltpu.VMEM(((Bb + 1) * p, 128), jnp.int32),
                pltpu.VMEM((Bb, Hp), jnp.bfloat16),
                pltpu.VMEM((Bb, Hp), jnp.float32),
            ],
        ),
        compiler_params=pltpu.CompilerParams(
            dimension_semantics=("parallel",),
            vmem_limit_bytes=48 * 1024 * 1024,
        ),
    )(ids3, table, wcat, b_p, wlin_p, blin_p)

    return logits_pad[:B, :O], h_last_pad[:B, :H]
